# Initial kernel scaffold; baseline (speedup 1.0000x reference)
import jax
import jax.numpy as jnp
from jax.experimental import pallas as pl


def kernel(r_node, i_node, r_edge, elu1_r_node, elu1_i_node, elu1_r_edge, elu2_r_node, elu2_i_node, elu2_r_edge, W_rnode, W_inode, W_edge, conv_Wq, conv_We, conv_Wv, conv_a, conv_Wi, lin_W, lin_b, out_W, out_b, edge_index, d2d_edge_index, segment_ids):
    raise NotImplementedError("write your pallas kernel here")


# speedup vs baseline: 3.9998x; 3.9998x over previous
import functools

import jax
import jax.numpy as jnp
from jax import lax
from jax.experimental import pallas as pl
from jax.experimental.pallas import tpu as pltpu
from jax.experimental.pallas import tpu_sc as plsc

N = 10000
E = 320000
ED = 160000
G = 128
HID = 64
NODE_DIM = 128
LAYERS = 4
AGGW = 80

NC = 2
NS = 16
NW = NC * NS
EPW = E // NW
DPW = ED // NW
CE = 80
CD = 40
RPT = N // NS

BN = 1000
NBN = N // BN
BE = 16000
NBE = E // BE

_f32 = jnp.float32


def _zero_acc(acc_sh, zb, s, width):
    z16 = jnp.zeros((16,), _f32)

    @pl.loop(0, 48)
    def _zb(i):
        for j in range(width // 16):
            zb[i, pl.ds(16 * j, 16)] = z16

    for k in range(13):
        pltpu.sync_copy(zb, acc_sh.at[pl.ds(s * 624 + k * 48, 48)])

    @pl.when(s == NS - 1)
    def _():
        pltpu.sync_copy(zb.at[pl.ds(0, N - NS * 624)],
                        acc_sh.at[pl.ds(NS * 624, N - NS * 624)])


def _writeback(acc_sh, out_h, c, s):
    for cc in range(NC):
        @pl.when(c == cc)
        def _(cc=cc):
            for k in range(13):
                sl = pl.ds(s * 624 + k * 48, 48)
                pltpu.sync_copy(acc_sh.at[sl], out_h.at[cc, sl])

            @pl.when(s == NS - 1)
            def _():
                sl = pl.ds(NS * 624, N - NS * 624)
                pltpu.sync_copy(acc_sh.at[sl], out_h.at[cc, sl])


def _sc_r2r_body(hqv_h, ewe_h, esrc_h, edst_h, avec_h, agg_o,
                 agg_sh, avec_v, zb80, idxs, idxd, gath, eweb, msg):
    c = lax.axis_index("c")
    s = lax.axis_index("s")
    wid = s * NC + c

    _zero_acc(agg_sh, zb80, s, AGGW)
    pltpu.sync_copy(avec_h, avec_v)
    plsc.subcore_barrier()

    ebase = wid * EPW
    iota16 = lax.broadcasted_iota(jnp.int32, (16,), 0)
    av = [avec_v[pl.ds(16 * j, 16)] for j in range(HID // 16)]
    aks = [av[j][i] for j in range(HID // 16) for i in range(16)]

    @pl.loop(0, EPW // CE)
    def _r2r(ci):
        base = ebase + ci * CE
        pltpu.sync_copy(esrc_h.at[pl.ds(base, CE)], idxs)
        pltpu.sync_copy(edst_h.at[pl.ds(base, CE)], idxd)
        pltpu.sync_copy(ewe_h.at[pl.ds(base, CE)], eweb)
        pltpu.sync_copy(hqv_h.at[idxs], gath)

        @pl.loop(0, CE // 16)
        def _edges(g):
            base_e = 16 * g
            rows = base_e + iota16
            logits = jnp.zeros((16,), _f32)
            for k in range(HID):
                colk = jnp.full((16,), k, jnp.int32)
                qk = (plsc.load_gather(gath, [rows, colk])
                      + plsc.load_gather(eweb, [rows, colk]))
                lk = jnp.where(qk >= 0, qk, 0.01 * qk)
                logits = logits + lk * aks[k]
            zvec = jnp.exp(logits)
            for i in range(16):
                e = base_e + i
                zs = zvec[i]
                for j in range(HID // 16):
                    msg[e, pl.ds(16 * j, 16)] = (
                        zs * gath[e, pl.ds(HID + 16 * j, 16)])
                msg[e, pl.ds(HID, 16)] = jnp.where(iota16 == 0, zs, 0.0)

        pltpu.sync_copy(msg, agg_sh.at[idxd], add=True)

    plsc.subcore_barrier()
    _writeback(agg_sh, agg_o, c, s)


def _sc_d2d_body(hiw_h, dsrc_h, ddst_h, tok_h, hia_o,
                 hia_sh, zb64, didxs, didxd, dgath, dcomp):
    del tok_h
    c = lax.axis_index("c")
    s = lax.axis_index("s")
    wid = s * NC + c

    _zero_acc(hia_sh, zb64, s, HID)
    plsc.subcore_barrier()

    dbase = wid * DPW

    @pl.loop(0, DPW // CD)
    def _d2d(ci):
        base = dbase + ci * CD
        pltpu.sync_copy(dsrc_h.at[pl.ds(base, CD)], didxs)
        pltpu.sync_copy(ddst_h.at[pl.ds(base, CD)], didxd)
        pltpu.sync_copy(hiw_h.at[didxs], dgath)

        @pl.loop(0, CD, unroll=4)
        def _cp(e):
            for j in range(HID // 16):
                dcomp[e, pl.ds(16 * j, 16)] = dgath[e, pl.ds(16 * j, 16)]

        pltpu.sync_copy(dcomp, hia_sh.at[didxd], add=True)

    plsc.subcore_barrier()
    _writeback(hia_sh, hia_o, c, s)


def _sc_mesh():
    return plsc.VectorSubcoreMesh(core_axis_name="c", subcore_axis_name="s",
                                  num_cores=NC, num_subcores=NS)


@functools.cache
def _get_sc_r2r():
  return pl.kernel(
    _sc_r2r_body,
    out_type=jax.ShapeDtypeStruct((NC, N, AGGW), _f32),
    mesh=_sc_mesh(),
    compiler_params=pltpu.CompilerParams(needs_layout_passes=False),
    scratch_types=[
        pltpu.VMEM_SHARED((N, AGGW), _f32),
        pltpu.VMEM((HID,), _f32),
        pltpu.VMEM((48, AGGW), _f32),
        pltpu.VMEM((CE,), jnp.int32),
        pltpu.VMEM((CE,), jnp.int32),
        pltpu.VMEM((CE, 2 * HID), _f32),
        pltpu.VMEM((CE, HID), _f32),
        pltpu.VMEM((CE, AGGW), _f32),
    ],
  )


@functools.cache
def _get_sc_d2d():
  return pl.kernel(
    _sc_d2d_body,
    out_type=jax.ShapeDtypeStruct((NC, N, HID), _f32),
    mesh=_sc_mesh(),
    compiler_params=pltpu.CompilerParams(needs_layout_passes=False),
    scratch_types=[
        pltpu.VMEM_SHARED((N, HID), _f32),
        pltpu.VMEM((48, HID), _f32),
        pltpu.VMEM((CD,), jnp.int32),
        pltpu.VMEM((CD,), jnp.int32),
        pltpu.VMEM((CD, 2 * HID), _f32),
        pltpu.VMEM((CD, HID), _f32),
    ],
  )


def _dot(a, b):
    return jnp.dot(a, b, preferred_element_type=_f32)


def _leaky(x):
    return jnp.where(x >= 0, x, 0.01 * x)


def _node_spec():
    return pl.BlockSpec((BN, HID), lambda i: (i, 0))


def _w_spec():
    return pl.BlockSpec((HID, HID), lambda i: (0, 0))


def _tc_pre0(r_node, i_node, W_rnode, W_inode, Wq, Wv, Wi):
    def body(rn, inode, wr, win, wq, wv, wgi, h_o, hi_o, hqv_o, hiw_o):
        h = _dot(rn[...], wr[...])
        hi = inode[...] * win[...]
        h_o[...] = h
        hi_o[...] = hi
        hqv_o[...] = jnp.concatenate([_dot(h, wq[...]), _dot(h, wv[...])],
                                     axis=1)
        hiw_o[...] = jnp.concatenate(
            [_dot(hi, wgi[...]), jnp.zeros((BN, HID), _f32)], axis=1)

    return pl.pallas_call(
        body,
        grid=(NBN,),
        in_specs=[
            pl.BlockSpec((BN, NODE_DIM), lambda i: (i, 0)),
            pl.BlockSpec((BN, 1), lambda i: (i, 0)),
            pl.BlockSpec((NODE_DIM, HID), lambda i: (0, 0)),
            pl.BlockSpec((1, HID), lambda i: (0, 0)),
            _w_spec(), _w_spec(), _w_spec(),
        ],
        out_specs=[_node_spec(), _node_spec(),
                   pl.BlockSpec((BN, 2 * HID), lambda i: (i, 0)),
                   pl.BlockSpec((BN, 2 * HID), lambda i: (i, 0))],
        out_shape=[jax.ShapeDtypeStruct((N, HID), _f32),
                   jax.ShapeDtypeStruct((N, HID), _f32),
                   jax.ShapeDtypeStruct((N, 2 * HID), _f32),
                   jax.ShapeDtypeStruct((N, 2 * HID), _f32)],
    )(r_node, i_node, W_rnode, W_inode, Wq, Wv, Wi)


def _tc_edge(r_edge, W_edge, We):
    def body(re_r, we_r, wl_r, o_r):
        m = _dot(we_r[...], wl_r[...])
        o_r[...] = _dot(re_r[...], m)

    return pl.pallas_call(
        body,
        grid=(NBE,),
        in_specs=[
            pl.BlockSpec((BE, 16), lambda i: (i, 0)),
            pl.BlockSpec((16, HID), lambda i: (0, 0)),
            _w_spec(),
        ],
        out_specs=pl.BlockSpec((BE, HID), lambda i: (i, 0)),
        out_shape=jax.ShapeDtypeStruct((E, HID), _f32),
    )(r_edge, W_edge, We)


def _post_block(h_r, hi_r, a0, a1, b0, b1):
    aggs = a0[...] + a1[...]
    aggv = aggs[:, :HID] / (aggs[:, HID:HID + 1] + 1e-9)
    hin = hi_r[...] + b0[...] + b1[...]
    hi2 = _leaky(hin)
    gate = jax.nn.sigmoid(hi2)
    h2 = _leaky(h_r[...] + aggv * gate)
    return h2, hi2


def _agg_spec():
    return pl.BlockSpec((BN, AGGW), lambda i: (i, 0))


def _tc_step(h, hi, agg0, agg1, hia0, hia1, Wq, Wv, Wi):
    def body(h_r, hi_r, a0, a1, b0, b1, wq, wv, wgi,
             h_o, hi_o, hqv_o, hiw_o):
        h2, hi2 = _post_block(h_r, hi_r, a0, a1, b0, b1)
        h_o[...] = h2
        hi_o[...] = hi2
        hqv_o[...] = jnp.concatenate([_dot(h2, wq[...]), _dot(h2, wv[...])],
                                     axis=1)
        hiw_o[...] = jnp.concatenate(
            [_dot(hi2, wgi[...]), jnp.zeros((BN, HID), _f32)], axis=1)

    return pl.pallas_call(
        body,
        grid=(NBN,),
        in_specs=[
            _node_spec(), _node_spec(),
            _agg_spec(), _agg_spec(),
            _node_spec(), _node_spec(),
            _w_spec(), _w_spec(), _w_spec(),
        ],
        out_specs=[_node_spec(), _node_spec(),
                   pl.BlockSpec((BN, 2 * HID), lambda i: (i, 0)),
                   pl.BlockSpec((BN, 2 * HID), lambda i: (i, 0))],
        out_shape=[jax.ShapeDtypeStruct((N, HID), _f32),
                   jax.ShapeDtypeStruct((N, HID), _f32),
                   jax.ShapeDtypeStruct((N, 2 * HID), _f32),
                   jax.ShapeDtypeStruct((N, 2 * HID), _f32)],
    )(h, hi, agg0, agg1, hia0, hia1, Wq, Wv, Wi)


def _tc_tail(h, hi, agg0, agg1, hia0, hia1, seg_f, lin_W, lin_b, out_W,
             out_b2):
    def body(h_r, hi_r, a0, a1, b0, b1, seg_r, lw_r, lb_r, ow_r, ob_r,
             out_r, acc):
        i = pl.program_id(0)

        @pl.when(i == 0)
        def _():
            acc[...] = jnp.zeros_like(acc)

        h2, _ = _post_block(h_r, hi_r, a0, a1, b0, b1)
        seg = seg_r[...]
        iota = lax.broadcasted_iota(jnp.int32, (BN, G), 1).astype(_f32)
        mask = (seg == iota).astype(_f32)
        acc[...] += lax.dot_general(mask, h2, (((0,), (0,)), ((), ())),
                                    preferred_element_type=_f32)

        @pl.when(i == NBN - 1)
        def _():
            x = acc[...]
            lw = lw_r[...]
            lb = lb_r[...]
            for t in range(3):
                x = jnp.maximum(_dot(x, lw[t]) + lb[t][None, :], 0.0)
            out_r[...] = _dot(x, ow_r[...]) + ob_r[...]

    return pl.pallas_call(
        body,
        grid=(NBN,),
        in_specs=[
            _node_spec(), _node_spec(),
            _agg_spec(), _agg_spec(),
            _node_spec(), _node_spec(),
            pl.BlockSpec((BN, 1), lambda i: (i, 0)),
            pl.BlockSpec((3, HID, HID), lambda i: (0, 0, 0)),
            pl.BlockSpec((3, HID), lambda i: (0, 0)),
            pl.BlockSpec((HID, 1), lambda i: (0, 0)),
            pl.BlockSpec((1, 1), lambda i: (0, 0)),
        ],
        out_specs=pl.BlockSpec((G, 1), lambda i: (0, 0)),
        out_shape=jax.ShapeDtypeStruct((G, 1), _f32),
        scratch_shapes=[pltpu.VMEM((G, HID), _f32)],
    )(h, hi, agg0, agg1, hia0, hia1, seg_f, lin_W, lin_b, out_W, out_b2)


def kernel(r_node, i_node, r_edge, elu1_r_node, elu1_i_node, elu1_r_edge,
           elu2_r_node, elu2_i_node, elu2_r_edge,
           W_rnode, W_inode, W_edge, conv_Wq, conv_We, conv_Wv, conv_a,
           conv_Wi, lin_W, lin_b, out_W, out_b,
           edge_index, d2d_edge_index, segment_ids):
    esrc = edge_index[0].astype(jnp.int32)
    edst = edge_index[1].astype(jnp.int32)
    dsrc = d2d_edge_index[0].astype(jnp.int32)
    ddst = d2d_edge_index[1].astype(jnp.int32)
    seg_f = segment_ids.astype(_f32)[:, None]
    out_b2 = out_b.reshape(1, 1)

    h, hi, hqv, hiw = _tc_pre0(r_node, i_node, W_rnode, W_inode,
                               conv_Wq[0], conv_Wv[0], conv_Wi[0])
    out = None
    for l in range(LAYERS):
        ewe = _tc_edge(r_edge, W_edge, conv_We[l])
        agg = _get_sc_r2r()(hqv, ewe, esrc, edst, conv_a[l])
        hia = _get_sc_d2d()(hiw, dsrc, ddst, agg)
        if l < LAYERS - 1:
            h, hi, hqv, hiw = _tc_step(h, hi, agg[0], agg[1], hia[0],
                                       hia[1], conv_Wq[l + 1],
                                       conv_Wv[l + 1], conv_Wi[l + 1])
        else:
            out = _tc_tail(h, hi, agg[0], agg[1], hia[0], hia[1], seg_f,
                           lin_W, lin_b, out_W, out_b2)
    return out
